# baseline (speedup 1.0000x reference)
import functools

import jax
import jax.numpy as jnp
from jax.experimental import pallas as pl
from jax.experimental.pallas import tpu as pltpu

BN_EPS = 1e-5
VMEM_LIMIT = 60000 * 1024


def _fused_kernel(x_ref, w1_ref, ws_ref, w2_ref, gamma_ref, beta_ref,
                  bout_ref, o_ref, g_ref, s_ref, scale_ref, shift_ref,
                  *, inv_b):
    p = pl.program_id(0)
    t = pl.program_id(1)
    bt = x_ref.shape[0]

    @pl.when(jnp.logical_and(p == 0, t == 0))
    def _():
        g_ref[...] = jnp.zeros_like(g_ref)
        s_ref[...] = jnp.zeros_like(s_ref)

    @pl.when(p == 0)
    def _():
        xf = x_ref[...]
        g_ref[...] += jax.lax.dot_general(
            xf, xf, (((0,), (0,)), ((), ())),
            preferred_element_type=jnp.float32)
        ones = jnp.ones((8, bt), jnp.float32)
        s_ref[...] += jnp.dot(ones, xf, preferred_element_type=jnp.float32)

    @pl.when(jnp.logical_and(p == 1, t == 0))
    def _():
        w1 = w1_ref[...]
        h = jnp.dot(g_ref[...], w1, preferred_element_type=jnp.float32)
        sq = jnp.sum(w1 * h, axis=0, keepdims=True)
        mean = jnp.dot(s_ref[...], w1,
                       preferred_element_type=jnp.float32)[0:1] * inv_b
        var = jnp.maximum(sq * inv_b - mean * mean, 0.0)
        scale = gamma_ref[...] * jax.lax.rsqrt(var + BN_EPS)
        scale_ref[...] = scale
        shift_ref[...] = beta_ref[...] - mean * scale

    @pl.when(p == 1)
    def _():
        xf = x_ref[...]
        y1 = jnp.dot(xf, w1_ref[...], preferred_element_type=jnp.float32)
        y_relu = jnp.maximum(y1 * scale_ref[...] + shift_ref[...], 0.0)
        y2 = jnp.dot(y_relu, w2_ref[...], preferred_element_type=jnp.float32)
        ys = jnp.dot(xf, ws_ref[...], preferred_element_type=jnp.float32)
        o_ref[...] = y2 + ys + bout_ref[...]


def kernel(x, w1t, b1, gamma, beta, w2t, b2, wst, bs):
    B, n_in = x.shape
    n_units = w1t.shape[1]
    del b1

    bt = min(2048, B)
    assert B % bt == 0 and n_in % 128 == 0 and n_units % 128 == 0
    tpc = B // bt
    inv_b = 1.0 / B
    bout = (b2 + bs).astype(jnp.float32)

    const = lambda p, t: (0, 0)
    out = pl.pallas_call(
        functools.partial(_fused_kernel, inv_b=inv_b),
        out_shape=jax.ShapeDtypeStruct((B, n_units), jnp.float32),
        grid=(2, tpc),
        in_specs=[
            pl.BlockSpec((bt, n_in), lambda p, t: (t, 0)),
            pl.BlockSpec((n_in, n_units), const),
            pl.BlockSpec((n_in, n_units), const),
            pl.BlockSpec((n_units, n_units), const),
            pl.BlockSpec((1, n_units), const),
            pl.BlockSpec((1, n_units), const),
            pl.BlockSpec((1, n_units), const),
        ],
        out_specs=pl.BlockSpec((bt, n_units), lambda p, t: (p * t, 0)),
        scratch_shapes=[
            pltpu.VMEM((n_in, n_in), jnp.float32),
            pltpu.VMEM((8, n_in), jnp.float32),
            pltpu.VMEM((1, n_units), jnp.float32),
            pltpu.VMEM((1, n_units), jnp.float32),
        ],
        compiler_params=pltpu.CompilerParams(
            dimension_semantics=("arbitrary", "arbitrary"),
            vmem_limit_bytes=VMEM_LIMIT),
    )(x, w1t, wst, w2t, gamma.astype(jnp.float32), beta.astype(jnp.float32),
      bout)

    return out


# speedup vs baseline: 1.0034x; 1.0034x over previous
import functools

import jax
import jax.numpy as jnp
from jax.experimental import pallas as pl
from jax.experimental.pallas import tpu as pltpu

BN_EPS = 1e-5
VMEM_LIMIT = 60000 * 1024


def _fused_kernel(x_ref, w1_ref, ws_ref, w2_ref, gamma_ref, beta_ref,
                  bout_ref, o_ref, g_ref, s_ref, scale_ref, shift_ref,
                  *, inv_b):
    p = pl.program_id(0)
    t = pl.program_id(1)
    bt = x_ref.shape[0]

    @pl.when(jnp.logical_and(p == 0, t == 0))
    def _():
        g_ref[...] = jnp.zeros_like(g_ref)
        s_ref[...] = jnp.zeros_like(s_ref)

    @pl.when(p == 0)
    def _():
        xf = x_ref[...]
        g_ref[...] += jax.lax.dot_general(
            xf, xf, (((0,), (0,)), ((), ())),
            preferred_element_type=jnp.float32)
        ones = jnp.ones((8, bt), jnp.float32)
        s_ref[...] += jnp.dot(ones, xf, preferred_element_type=jnp.float32)

    @pl.when(jnp.logical_and(p == 1, t == 0))
    def _():
        w1 = w1_ref[...]
        h = jnp.dot(g_ref[...], w1, preferred_element_type=jnp.float32)
        sq = jnp.sum(w1 * h, axis=0, keepdims=True)
        mean = jnp.dot(s_ref[...], w1,
                       preferred_element_type=jnp.float32)[0:1] * inv_b
        var = jnp.maximum(sq * inv_b - mean * mean, 0.0)
        scale = gamma_ref[...] * jax.lax.rsqrt(var + BN_EPS)
        scale_ref[...] = scale
        shift_ref[...] = beta_ref[...] - mean * scale

    @pl.when(p == 1)
    def _():
        xf = x_ref[...]
        y1 = jnp.dot(xf, w1_ref[...], preferred_element_type=jnp.float32)
        ys = jnp.dot(xf, ws_ref[...], preferred_element_type=jnp.float32)
        y_relu = jnp.maximum(y1 * scale_ref[...] + shift_ref[...], 0.0)
        y2 = jnp.dot(y_relu, w2_ref[...], preferred_element_type=jnp.float32)
        o_ref[...] = y2 + ys + bout_ref[...]


def kernel(x, w1t, b1, gamma, beta, w2t, b2, wst, bs):
    B, n_in = x.shape
    n_units = w1t.shape[1]
    del b1

    bt = min(2048, B)
    assert B % bt == 0 and n_in % 128 == 0 and n_units % 128 == 0
    tpc = B // bt
    inv_b = 1.0 / B
    bout = (b2 + bs).astype(jnp.float32)

    const = lambda p, t: (0, 0)
    out = pl.pallas_call(
        functools.partial(_fused_kernel, inv_b=inv_b),
        out_shape=jax.ShapeDtypeStruct((B, n_units), jnp.float32),
        grid=(2, tpc),
        in_specs=[
            pl.BlockSpec((bt, n_in), lambda p, t: (t, 0)),
            pl.BlockSpec((n_in, n_units), const),
            pl.BlockSpec((n_in, n_units), const),
            pl.BlockSpec((n_units, n_units), const),
            pl.BlockSpec((1, n_units), const),
            pl.BlockSpec((1, n_units), const),
            pl.BlockSpec((1, n_units), const),
        ],
        out_specs=pl.BlockSpec((bt, n_units), lambda p, t: (p * t, 0)),
        scratch_shapes=[
            pltpu.VMEM((n_in, n_in), jnp.float32),
            pltpu.VMEM((8, n_in), jnp.float32),
            pltpu.VMEM((1, n_units), jnp.float32),
            pltpu.VMEM((1, n_units), jnp.float32),
        ],
        compiler_params=pltpu.CompilerParams(
            dimension_semantics=("arbitrary", "arbitrary"),
            vmem_limit_bytes=VMEM_LIMIT),
    )(x, w1t, wst, w2t, gamma.astype(jnp.float32), beta.astype(jnp.float32),
      bout)

    return out


# speedup vs baseline: 1.0215x; 1.0180x over previous
import functools

import jax
import jax.numpy as jnp
from jax.experimental import pallas as pl
from jax.experimental.pallas import tpu as pltpu

BN_EPS = 1e-5
VMEM_LIMIT = 60000 * 1024


def _fused_kernel(x_ref, w1_ref, ws_ref, w2_ref, gamma_ref, beta_ref,
                  b2_ref, bs_ref, o_ref, g_ref, s_ref, scale_ref, shift_ref,
                  *, inv_b):
    p = pl.program_id(0)
    t = pl.program_id(1)
    bt = x_ref.shape[0]

    @pl.when(jnp.logical_and(p == 0, t == 0))
    def _():
        g_ref[...] = jnp.zeros_like(g_ref)
        s_ref[...] = jnp.zeros_like(s_ref)

    @pl.when(p == 0)
    def _():
        xf = x_ref[...]
        g_ref[...] += jax.lax.dot_general(
            xf, xf, (((0,), (0,)), ((), ())),
            preferred_element_type=jnp.float32)
        ones = jnp.ones((8, bt), jnp.float32)
        s_ref[...] += jnp.dot(ones, xf, preferred_element_type=jnp.float32)

    @pl.when(jnp.logical_and(p == 1, t == 0))
    def _():
        w1 = w1_ref[...]
        h = jnp.dot(g_ref[...], w1, preferred_element_type=jnp.float32)
        sq = jnp.sum(w1 * h, axis=0, keepdims=True)
        mean = jnp.dot(s_ref[...], w1,
                       preferred_element_type=jnp.float32)[0:1] * inv_b
        var = jnp.maximum(sq * inv_b - mean * mean, 0.0)
        scale = gamma_ref[...] * jax.lax.rsqrt(var + BN_EPS)
        scale_ref[...] = scale
        shift_ref[...] = beta_ref[...] - mean * scale

    @pl.when(p == 1)
    def _():
        xf = x_ref[...]
        y1 = jnp.dot(xf, w1_ref[...], preferred_element_type=jnp.float32)
        ys = jnp.dot(xf, ws_ref[...], preferred_element_type=jnp.float32)
        y_relu = jnp.maximum(y1 * scale_ref[...] + shift_ref[...], 0.0)
        y2 = jnp.dot(y_relu, w2_ref[...], preferred_element_type=jnp.float32)
        o_ref[...] = y2 + ys + (b2_ref[...] + bs_ref[...])


def kernel(x, w1t, b1, gamma, beta, w2t, b2, wst, bs):
    B, n_in = x.shape
    n_units = w1t.shape[1]
    del b1

    bt = min(2048, B)
    assert B % bt == 0 and n_in % 128 == 0 and n_units % 128 == 0
    tpc = B // bt
    inv_b = 1.0 / B

    const = lambda p, t: (0, 0)
    out = pl.pallas_call(
        functools.partial(_fused_kernel, inv_b=inv_b),
        out_shape=jax.ShapeDtypeStruct((B, n_units), jnp.float32),
        grid=(2, tpc),
        in_specs=[
            pl.BlockSpec((bt, n_in), lambda p, t: (t, 0)),
            pl.BlockSpec((n_in, n_units), const),
            pl.BlockSpec((n_in, n_units), const),
            pl.BlockSpec((n_units, n_units), const),
            pl.BlockSpec((1, n_units), const),
            pl.BlockSpec((1, n_units), const),
            pl.BlockSpec((1, n_units), const),
            pl.BlockSpec((1, n_units), const),
        ],
        out_specs=pl.BlockSpec((bt, n_units), lambda p, t: (p * t, 0)),
        scratch_shapes=[
            pltpu.VMEM((n_in, n_in), jnp.float32),
            pltpu.VMEM((8, n_in), jnp.float32),
            pltpu.VMEM((1, n_units), jnp.float32),
            pltpu.VMEM((1, n_units), jnp.float32),
        ],
        compiler_params=pltpu.CompilerParams(
            dimension_semantics=("arbitrary", "arbitrary"),
            vmem_limit_bytes=VMEM_LIMIT),
    )(x, w1t, wst, w2t, gamma.astype(jnp.float32), beta.astype(jnp.float32),
      b2.astype(jnp.float32), bs.astype(jnp.float32))

    return out
